# baseline (speedup 1.0000x reference)
import functools

import jax
import jax.numpy as jnp
from jax import lax
from jax.experimental import pallas as pl
from jax.experimental.pallas import tpu as pltpu
from jax.experimental.pallas import tpu_sc as plsc


@functools.lru_cache(maxsize=None)
def _build_emb(B: int, D: int, V: int):
    info = plsc.get_sparse_core_info()
    NC, NS = info.num_cores, info.num_subcores
    NW = NC * NS
    assert B % NW == 0
    bpw = B // NW
    C = 32
    while bpw % (2 * C):
        C //= 2
    nouter = bpw // (2 * C)
    mesh = plsc.VectorSubcoreMesh(core_axis_name="c", subcore_axis_name="s")

    @functools.partial(
        pl.kernel,
        mesh=mesh,
        out_type=jax.ShapeDtypeStruct((B, D), jnp.float32),
        scratch_types=[
            pltpu.VMEM((bpw,), jnp.int32),
            pltpu.VMEM((C, D), jnp.float32),
            pltpu.VMEM((C, D), jnp.float32),
            pltpu.SemaphoreType.DMA,
            pltpu.SemaphoreType.DMA,
            pltpu.SemaphoreType.DMA,
            pltpu.SemaphoreType.DMA,
        ],
    )
    def emb(idx_hbm, table_hbm, out_hbm, idx_v, rows0, rows1, g0, g1, s0, s1):
        wid = lax.axis_index("s") * NC + lax.axis_index("c")
        base = wid * bpw
        pltpu.sync_copy(idx_hbm.at[pl.ds(base, bpw)], idx_v)

        def g_start(i, buf, sem):
            pltpu.async_copy(table_hbm.at[idx_v.at[pl.ds(i * C, C)]], buf, sem)

        def g_wait(buf, sem):
            pltpu.make_async_copy(
                table_hbm.at[idx_v.at[pl.ds(0, C)]], buf, sem
            ).wait()

        def s_start(i, buf, sem):
            pltpu.async_copy(buf, out_hbm.at[pl.ds(base + i * C, C)], sem)

        def s_wait(buf, sem):
            pltpu.make_async_copy(buf, out_hbm.at[pl.ds(base, C)], sem).wait()

        g_start(0, rows0, g0)

        def body(o, carry):
            i0 = 2 * o

            @pl.when(o > 0)
            def _():
                s_wait(rows1, s1)

            g_start(i0 + 1, rows1, g1)
            g_wait(rows0, g0)
            s_start(i0, rows0, s0)
            s_wait(rows0, s0)

            @pl.when(o + 1 < nouter)
            def _():
                g_start(i0 + 2, rows0, g0)

            g_wait(rows1, g1)
            s_start(i0 + 1, rows1, s1)
            return carry

        lax.fori_loop(0, nouter, body, 0)
        s_wait(rows1, s1)

    return emb


def kernel(seqs, W_embed):
    batch, seq = seqs.shape
    V, D = W_embed.shape
    idx = seqs.reshape(-1).astype(jnp.int32)
    emb = _build_emb(batch * seq, D, V)
    out = emb(idx, W_embed)
    return out.reshape(batch, seq, D)
